# baseline (speedup 1.0000x reference)
import functools
import math

import jax
import jax.numpy as jnp
from jax import lax
from jax.experimental import pallas as pl
from jax.experimental.pallas import tpu as pltpu
from jax.experimental.pallas import tpu_sc as plsc

D = 64
SCALE = math.sqrt(D)
LB = 128
LANES = 16


def _make_sc_kernel(S: int, B: int, NC: int, NS: int):
  NW = NC * NS
  assert B == NW * LB and S % 2 == 0

  mesh = plsc.VectorSubcoreMesh(core_axis_name="c", subcore_axis_name="s")

  @functools.partial(
      pl.kernel,
      out_type=jax.ShapeDtypeStruct((S, D, B), jnp.float32),
      mesh=mesh,
      compiler_params=pltpu.CompilerParams(
          needs_layout_passes=False, disable_bounds_checks=True),
      scratch_types=[
          pltpu.VMEM((S, LB), jnp.int32),
          pltpu.VMEM((4, LB), jnp.int32),
          pltpu.VMEM((4, LB), jnp.int32),
          [pltpu.VMEM((LB, LB), jnp.float32) for _ in range(4)],
          [pltpu.VMEM((D, LB), jnp.float32) for _ in range(4)],
          [pltpu.SemaphoreType.DMA for _ in range(4)],
          [pltpu.SemaphoreType.DMA for _ in range(4)],
      ],
  )
  def k(idx_hbm, tab_hbm, out_hbm, idxcol, gidx, poff,
        rbufs, obufs, gsems, osems):
    NB = 4
    cid = lax.axis_index("c")
    sid = lax.axis_index("s")
    wid = sid * NC + cid
    lane0 = wid * LB

    pltpu.sync_copy(idx_hbm.at[:, pl.ds(lane0, LB)], idxcol)

    def prep(s, b):
      for j in range(LB // LANES):
        v = idxcol[s, pl.ds(j * LANES, LANES)]
        gidx[b, pl.ds(j * LANES, LANES)] = lax.shift_right_logical(v, 1)
        poff[b, pl.ds(j * LANES, LANES)] = lax.shift_left(
            lax.bitwise_and(v, 1), 6)
      pltpu.async_copy(tab_hbm.at[gidx.at[b]], rbufs[b], gsems[b])

    def drain_gather(b):
      pltpu.make_async_copy(tab_hbm.at[gidx.at[b]], rbufs[b], gsems[b]).wait()

    def transpose_scale(s, b):
      rbuf, obuf = rbufs[b], obufs[b]
      iot = lax.iota(jnp.int32, LANES)

      @plsc.parallel_loop(0, LB // LANES, unroll=1)
      def _(jb):
        j0 = jb * LANES
        jvec = iot + j0
        poffv = poff[b, pl.ds(j0, LANES)]
        for db in range(D // LANES):
          cbase = poffv + (db * LANES)

          @plsc.parallel_loop(0, LANES, unroll=4)
          def _(r):
            rot = lax.bitwise_and(iot + r, LANES - 1)
            vals = plsc.load_gather(rbuf, [jvec, cbase + rot])
            plsc.store_scatter(obuf, [rot + (db * LANES), jvec], vals * SCALE)

    def write(s, b):
      pltpu.async_copy(obufs[b], out_hbm.at[s, :, pl.ds(lane0, LB)], osems[b])

    def drain_write(s, b):
      pltpu.make_async_copy(
          obufs[b], out_hbm.at[s, :, pl.ds(lane0, LB)], osems[b]).wait()

    for s0 in range(NB - 1):
      prep(s0, s0)
    for c in range(NB):
      prep(c + NB - 1, (c + NB - 1) % NB)
      drain_gather(c % NB)
      transpose_scale(c, c % NB)
      write(c, c % NB)

    @pl.loop(NB, S - NB, step=NB)
    def _(c0):
      for b in range(NB):
        c = c0 + b
        m = b
        f = (b + NB - 1) % NB
        prep(c + NB - 1, f)
        drain_gather(m)
        drain_write(c - NB, m)
        transpose_scale(c, m)
        write(c, m)

    prep(S - 1, (S - 1) % NB)
    for c in range(S - NB, S):
      m = c % NB
      drain_gather(m)
      drain_write(c - NB, m)
      transpose_scale(c, m)
      write(c, m)
    for c in range(S - NB, S):
      drain_write(c, c % NB)

  return k


def kernel(indices, table):
  B0, S = indices.shape
  V = table.shape[0]
  info = plsc.get_sparse_core_info()
  NC, NS = info.num_cores, info.num_subcores
  idx_t = indices.astype(jnp.int32).T
  tpair = table.reshape(V // 2, 2 * D)
  out_t = _make_sc_kernel(S, B0, NC, NS)(idx_t, tpair)
  return jnp.transpose(out_t, (2, 0, 1))


# speedup vs baseline: 1.1117x; 1.1117x over previous
import functools
import math

import jax
import jax.numpy as jnp
from jax import lax
from jax.experimental import pallas as pl
from jax.experimental.pallas import tpu as pltpu
from jax.experimental.pallas import tpu_sc as plsc

D = 64
SCALE = math.sqrt(D)
LB = 128
LANES = 16


def _make_sc_kernel(S: int, B: int, NC: int, NS: int):
  NW = NC * NS
  assert B == NW * LB and S % 2 == 0

  mesh = plsc.VectorSubcoreMesh(core_axis_name="c", subcore_axis_name="s")

  @functools.partial(
      pl.kernel,
      out_type=jax.ShapeDtypeStruct((S, D, B), jnp.float32),
      mesh=mesh,
      compiler_params=pltpu.CompilerParams(
          needs_layout_passes=False, disable_bounds_checks=True),
      scratch_types=[
          pltpu.VMEM((S, LB), jnp.int32),
          pltpu.VMEM((4, LB), jnp.int32),
          [pltpu.VMEM((LB, LB), jnp.float32) for _ in range(4)],
          [pltpu.VMEM((D, LB), jnp.float32) for _ in range(4)],
          [pltpu.SemaphoreType.DMA for _ in range(4)],
          [pltpu.SemaphoreType.DMA for _ in range(4)],
      ],
  )
  def k(idx_hbm, tab_hbm, out_hbm, idxcol, gidx,
        rbufs, obufs, gsems, osems):
    NB = 4
    cid = lax.axis_index("c")
    sid = lax.axis_index("s")
    wid = sid * NC + cid
    lane0 = wid * LB

    pltpu.sync_copy(idx_hbm.at[:, pl.ds(lane0, LB)], idxcol)

    def prep(s, b):
      for j in range(LB // LANES):
        gidx[b, pl.ds(j * LANES, LANES)] = idxcol[s, pl.ds(j * LANES, LANES)]
      pltpu.async_copy(tab_hbm.at[gidx.at[b]], rbufs[b], gsems[b])

    def drain_gather(b):
      pltpu.make_async_copy(tab_hbm.at[gidx.at[b]], rbufs[b], gsems[b]).wait()

    def transpose_scale(s, b):
      rbuf, obuf = rbufs[b], obufs[b]
      iot = lax.iota(jnp.int32, LANES)

      @plsc.parallel_loop(0, LB // LANES, unroll=1)
      def _(jb):
        j0 = jb * LANES
        jvec = iot + j0
        for db in range(D // LANES):
          cbase = jnp.full((LANES,), db * LANES, jnp.int32)

          @plsc.parallel_loop(0, LANES, unroll=4)
          def _(r):
            rot = lax.bitwise_and(iot + r, LANES - 1)
            vals = plsc.load_gather(rbuf, [jvec, cbase + rot])
            plsc.store_scatter(obuf, [rot + (db * LANES), jvec], vals * SCALE)

    def write(s, b):
      pltpu.async_copy(obufs[b], out_hbm.at[s, :, pl.ds(lane0, LB)], osems[b])

    def drain_write(s, b):
      pltpu.make_async_copy(
          obufs[b], out_hbm.at[s, :, pl.ds(lane0, LB)], osems[b]).wait()

    for s0 in range(NB - 1):
      prep(s0, s0)
    for c in range(NB):
      prep(c + NB - 1, (c + NB - 1) % NB)
      drain_gather(c % NB)
      transpose_scale(c, c % NB)
      write(c, c % NB)

    @pl.loop(NB, S - NB, step=NB)
    def _(c0):
      for b in range(NB):
        c = c0 + b
        m = b
        f = (b + NB - 1) % NB
        prep(c + NB - 1, f)
        drain_gather(m)
        drain_write(c - NB, m)
        transpose_scale(c, m)
        write(c, m)

    prep(S - 1, (S - 1) % NB)
    for c in range(S - NB, S):
      m = c % NB
      drain_gather(m)
      drain_write(c - NB, m)
      transpose_scale(c, m)
      write(c, m)
    for c in range(S - NB, S):
      drain_write(c, c % NB)

  return k


def kernel(indices, table):
  B0, S = indices.shape
  V = table.shape[0]
  info = plsc.get_sparse_core_info()
  NC, NS = info.num_cores, info.num_subcores
  idx_t = indices.astype(jnp.int32).T
  tpad = jnp.pad(table, ((0, 0), (0, D)))
  out_t = _make_sc_kernel(S, B0, NC, NS)(idx_t, tpad)
  return jnp.transpose(out_t, (2, 0, 1))


# speedup vs baseline: 1.7070x; 1.5354x over previous
import functools
import math

import jax
import jax.numpy as jnp
from jax import lax
from jax.experimental import pallas as pl
from jax.experimental.pallas import tpu as pltpu
from jax.experimental.pallas import tpu_sc as plsc

D = 64
SCALE = math.sqrt(D)
LB = 128
LANES = 16


def _make_sc_kernel(S: int, B: int, NC: int, NS: int):
  NW = NC * NS
  assert B == NW * LB and S % 2 == 0

  mesh = plsc.VectorSubcoreMesh(core_axis_name="c", subcore_axis_name="s")

  @functools.partial(
      pl.kernel,
      out_type=jax.ShapeDtypeStruct((S, D, B), jnp.float32),
      mesh=mesh,
      compiler_params=pltpu.CompilerParams(
          needs_layout_passes=False, disable_bounds_checks=True),
      scratch_types=[
          pltpu.VMEM((S, LB), jnp.int32),
          pltpu.VMEM((4, LB), jnp.int32),
          pltpu.VMEM((4, LB), jnp.int32),
          [pltpu.VMEM((LB, LB), jnp.float32) for _ in range(4)],
          [pltpu.VMEM((D, LB), jnp.float32) for _ in range(4)],
          [pltpu.SemaphoreType.DMA for _ in range(4)],
          [pltpu.SemaphoreType.DMA for _ in range(4)],
      ],
  )
  def k(idx_hbm, tab_hbm, out_hbm, idxcol, gidx, poff,
        rbufs, obufs, gsems, osems):
    NB = 4
    cid = lax.axis_index("c")
    sid = lax.axis_index("s")
    wid = sid * NC + cid
    lane0 = wid * LB

    pltpu.sync_copy(idx_hbm.at[:, pl.ds(lane0, LB)], idxcol)

    def prep(s, b):
      for j in range(LB // LANES):
        v = idxcol[s, pl.ds(j * LANES, LANES)]
        gidx[b, pl.ds(j * LANES, LANES)] = lax.shift_right_logical(v, 1)
        poff[b, pl.ds(j * LANES, LANES)] = lax.shift_left(
            lax.bitwise_and(v, 1), 6)
      pltpu.async_copy(tab_hbm.at[gidx.at[b]], rbufs[b], gsems[b])

    def drain_gather(b):
      pltpu.make_async_copy(tab_hbm.at[gidx.at[b]], rbufs[b], gsems[b]).wait()

    def transpose_scale(s, b):
      rbuf, obuf = rbufs[b], obufs[b]
      iot = lax.iota(jnp.int32, LANES)

      @plsc.parallel_loop(0, LB // LANES, unroll=1)
      def _(jb):
        j0 = jb * LANES
        jvec = iot + j0
        poffv = poff[b, pl.ds(j0, LANES)]
        for db in range(D // LANES):
          cbase = poffv + (db * LANES)

          @plsc.parallel_loop(0, LANES, unroll=4)
          def _(r):
            rot = lax.bitwise_and(iot + r, LANES - 1)
            vals = plsc.load_gather(rbuf, [jvec, cbase + rot])
            plsc.store_scatter(obuf, [rot + (db * LANES), jvec], vals * SCALE)

    def write(s, b):
      pltpu.async_copy(obufs[b], out_hbm.at[s, :, pl.ds(lane0, LB)], osems[b])

    def drain_write(s, b):
      pltpu.make_async_copy(
          obufs[b], out_hbm.at[s, :, pl.ds(lane0, LB)], osems[b]).wait()

    for s0 in range(NB - 1):
      prep(s0, s0)
    for c in range(NB):
      prep(c + NB - 1, (c + NB - 1) % NB)
      drain_gather(c % NB)
      transpose_scale(c, c % NB)
      write(c, c % NB)

    @pl.loop(NB, S - NB, step=NB)
    def _(c0):
      for b in range(NB):
        c = c0 + b
        m = b
        f = (b + NB - 1) % NB
        prep(c + NB - 1, f)
        drain_gather(m)
        drain_write(c - NB, m)
        transpose_scale(c, m)
        write(c, m)

    prep(S - 1, (S - 1) % NB)
    for c in range(S - NB, S):
      m = c % NB
      drain_gather(m)
      drain_write(c - NB, m)
      transpose_scale(c, m)
      write(c, m)
    for c in range(S - NB, S):
      drain_write(c, c % NB)

  return k


def _make_prepass(V: int, NC: int, NS: int):
  NW = NC * NS
  W = V // LB
  TAIL = (V - W * LB) // 2
  FULL_T = W // NW
  mesh = plsc.VectorSubcoreMesh(core_axis_name="c", subcore_axis_name="s")
  NB = 4

  @functools.partial(
      pl.kernel,
      out_type=jax.ShapeDtypeStruct((V // 2, LB), jnp.float32),
      mesh=mesh,
      compiler_params=pltpu.CompilerParams(
          needs_layout_passes=False, disable_bounds_checks=True),
      scratch_types=[
          [pltpu.VMEM((D, LB), jnp.float32) for _ in range(NB)],
          [pltpu.VMEM((D, LB), jnp.float32) for _ in range(NB)],
          pltpu.VMEM((D, D), jnp.float32),
          pltpu.VMEM((D // 2, LB), jnp.float32),
          [pltpu.SemaphoreType.DMA for _ in range(NB)],
          [pltpu.SemaphoreType.DMA for _ in range(NB)],
          pltpu.SemaphoreType.DMA,
      ],
  )
  def k(tnat_hbm, out_hbm, pbufs, obufs, psp, osp, isems, osems, tsem):
    cid = lax.axis_index("c")
    sid = lax.axis_index("s")
    wid = sid * NC + cid
    iot = lax.iota(jnp.int32, LANES)

    def win_of(t):
      return wid + t * NW

    def fire(t, b):
      pltpu.async_copy(
          tnat_hbm.at[:, pl.ds(win_of(t) * LB, LB)], pbufs[b], isems[b])

    def drain_in(t, b):
      pltpu.make_async_copy(
          tnat_hbm.at[:, pl.ds(win_of(t) * LB, LB)], pbufs[b], isems[b]).wait()

    def transpose_panel(pbuf, obuf, nq):
      @pl.loop(0, nq)
      def _(qi):
        q0 = qi * LANES
        c2base = iot * 2 + (2 * q0)
        qvec = iot + q0

        @pl.loop(0, LB // LANES)
        def _(cbi):
          cb = cbi * LANES
          pcol = c2base + lax.shift_right_logical(cbi, 2)
          rowb = lax.shift_left(lax.bitwise_and(cbi, 3), 4)

          @plsc.parallel_loop(0, LANES, unroll=4)
          def _(r):
            rot = lax.bitwise_and(iot + r, LANES - 1)
            vals = plsc.load_gather(pbuf, [rowb + rot, pcol])
            plsc.store_scatter(obuf, [qvec, cb + rot], vals)

    def transpose(b):
      transpose_panel(pbufs[b], obufs[b], D // LANES)

    def write(t, b):
      pltpu.async_copy(
          obufs[b], out_hbm.at[pl.ds(win_of(t) * (LB // 2), LB // 2)],
          osems[b])

    def drain_write(t, b):
      pltpu.make_async_copy(
          obufs[b], out_hbm.at[pl.ds(win_of(t) * (LB // 2), LB // 2)],
          osems[b]).wait()

    RING = 240
    for t0 in range(NB - 1):
      fire(t0, t0)
    for t0 in range(NB):
      fire(t0 + NB - 1, (t0 + NB - 1) % NB)
      drain_in(t0, t0 % NB)
      transpose(t0 % NB)
      write(t0, t0 % NB)

    @pl.loop(NB, RING - NB, step=NB)
    def _(t0):
      for b in range(NB):
        tt = t0 + b
        f = (b + NB - 1) % NB
        fire(tt + NB - 1, f)
        drain_in(tt, b)
        drain_write(tt - NB, b)
        transpose(b)
        write(tt, b)

    fire(RING - 1, (RING - 1) % NB)
    for tt in range(RING - NB, RING):
      m = tt % NB
      drain_in(tt, m)
      drain_write(tt - NB, m)
      transpose(m)
      write(tt, m)
    for tt in range(RING - NB, RING):
      drain_write(tt, tt % NB)

    def one_window(tt):
      fire(tt, 0)
      drain_in(tt, 0)
      transpose(0)
      write(tt, 0)
      drain_write(tt, 0)

    for tt in range(RING, FULL_T):
      one_window(tt)

    @pl.when(wid < W - FULL_T * NW)
    def _():
      one_window(FULL_T)

    @pl.when(wid == W - FULL_T * NW)
    def _():
      pltpu.async_copy(tnat_hbm.at[:, pl.ds(W * LB, D)], psp, tsem)
      pltpu.make_async_copy(
          tnat_hbm.at[:, pl.ds(W * LB, D)], psp, tsem).wait()
      transpose_panel(psp, osp, D // (2 * LANES))
      pltpu.async_copy(osp, out_hbm.at[pl.ds(W * (LB // 2), TAIL)], tsem)
      pltpu.make_async_copy(
          osp, out_hbm.at[pl.ds(W * (LB // 2), TAIL)], tsem).wait()

  return k


def kernel(indices, table):
  B0, S = indices.shape
  V = table.shape[0]
  info = plsc.get_sparse_core_info()
  NC, NS = info.num_cores, info.num_subcores
  idx_t = indices.astype(jnp.int32).T
  tnat = table.T
  tpair = _make_prepass(V, NC, NS)(tnat)
  out_t = _make_sc_kernel(S, B0, NC, NS)(idx_t, tpair)
  return jnp.transpose(out_t, (2, 0, 1))
